# baseline (speedup 1.0000x reference)
import jax
import jax.numpy as jnp
from jax.experimental import pallas as pl
from jax.experimental.pallas import tpu as pltpu

BLK = 256


def _prep_kernel(a_ref, nf_ref, wt_ref, b_ref, abf_ref, dis_ref, x0_ref):
    a = a_ref[...]
    abf_ref[...] = a.astype(jnp.bfloat16)
    deg = jnp.sum(a, axis=1, keepdims=True) + 1.0
    deg = jnp.maximum(deg, 1.0)
    dis_ref[...] = jax.lax.rsqrt(deg)
    p = jax.lax.dot(nf_ref[...], wt_ref[...],
                    preferred_element_type=jnp.float32) + b_ref[...]
    x0_ref[...] = jnp.maximum(p, 0.0)


def _layer_kernel(a_ref, x_ref, dis_ref, wt_ref, b_ref, out_ref, yh_s, yl_s):
    i = pl.program_id(0)

    @pl.when(i == 0)
    def _():
        y = x_ref[...] * dis_ref[...]
        yh = y.astype(jnp.bfloat16)
        yh_s[...] = yh
        yl_s[...] = (y - yh.astype(jnp.float32)).astype(jnp.bfloat16)

    a = a_ref[...]
    m = jax.lax.dot(a, yh_s[...], preferred_element_type=jnp.float32)
    m += jax.lax.dot(a, yl_s[...], preferred_element_type=jnp.float32)
    xb = x_ref[pl.ds(i * BLK, BLK), :]
    db = dis_ref[pl.ds(i * BLK, BLK), :]
    m = db * m + (db * db) * xb
    xn = jnp.maximum(jax.lax.dot(m, wt_ref[...],
                                 preferred_element_type=jnp.float32)
                     + b_ref[...], 0.0)
    out_ref[...] = xb + xn


def kernel(node_features, adjacency_matrix, W_in, b_in, W0, b0, W1, b1, W2, b2):
    n = adjacency_matrix.shape[0]
    in_dim = node_features.shape[1]
    d = W_in.shape[0]
    nblk = n // BLK

    prep = pl.pallas_call(
        _prep_kernel,
        grid=(nblk,),
        in_specs=[
            pl.BlockSpec((BLK, n), lambda i: (i, 0)),
            pl.BlockSpec((BLK, in_dim), lambda i: (i, 0)),
            pl.BlockSpec((in_dim, d), lambda i: (0, 0)),
            pl.BlockSpec((1, d), lambda i: (0, 0)),
        ],
        out_specs=(
            pl.BlockSpec((BLK, n), lambda i: (i, 0)),
            pl.BlockSpec((BLK, 1), lambda i: (i, 0)),
            pl.BlockSpec((BLK, d), lambda i: (i, 0)),
        ),
        out_shape=(
            jax.ShapeDtypeStruct((n, n), jnp.bfloat16),
            jax.ShapeDtypeStruct((n, 1), jnp.float32),
            jax.ShapeDtypeStruct((n, d), jnp.float32),
        ),
    )
    abf, dis, x = prep(adjacency_matrix, node_features, W_in.T,
                       b_in.reshape(1, d))

    layer = pl.pallas_call(
        _layer_kernel,
        grid=(nblk,),
        in_specs=[
            pl.BlockSpec((BLK, n), lambda i: (i, 0)),
            pl.BlockSpec((n, d), lambda i: (0, 0)),
            pl.BlockSpec((n, 1), lambda i: (0, 0)),
            pl.BlockSpec((d, d), lambda i: (0, 0)),
            pl.BlockSpec((1, d), lambda i: (0, 0)),
        ],
        out_specs=pl.BlockSpec((BLK, d), lambda i: (i, 0)),
        out_shape=jax.ShapeDtypeStruct((n, d), jnp.float32),
        scratch_shapes=[
            pltpu.VMEM((n, d), jnp.bfloat16),
            pltpu.VMEM((n, d), jnp.bfloat16),
        ],
    )
    for W, b in ((W0, b0), (W1, b1), (W2, b2)):
        x = layer(abf, x, dis, W.T, b.reshape(1, d))
    return x


# speedup vs baseline: 1.1320x; 1.1320x over previous
import jax
import jax.numpy as jnp
from jax.experimental import pallas as pl
from jax.experimental.pallas import tpu as pltpu

BLK = 256


def _prep_kernel(a_ref, nf_ref, wt_ref, b_ref, abf_ref, dis_ref, x0_ref):
    a = a_ref[...]
    abf_ref[...] = a.astype(jnp.bfloat16)
    deg = jnp.sum(a, axis=1, keepdims=True) + 1.0
    deg = jnp.maximum(deg, 1.0)
    dis_ref[...] = jax.lax.rsqrt(deg)
    p = jax.lax.dot(nf_ref[...], wt_ref[...],
                    preferred_element_type=jnp.float32) + b_ref[...]
    x0_ref[...] = jnp.maximum(p, 0.0)


def _layer_kernel(a_ref, x_ref, dis_ref, wt_ref, b_ref, out_ref, yh_s, yl_s):
    i = pl.program_id(0)

    @pl.when(i == 0)
    def _():
        y = x_ref[...] * dis_ref[...]
        yh = y.astype(jnp.bfloat16)
        yh_s[...] = yh
        yl_s[...] = (y - yh.astype(jnp.float32)).astype(jnp.bfloat16)

    a = a_ref[...]
    m = jax.lax.dot(a, yh_s[...], preferred_element_type=jnp.float32)
    xb = x_ref[pl.ds(i * BLK, BLK), :]
    db = dis_ref[pl.ds(i * BLK, BLK), :]
    m = db * m + (db * db) * xb
    xn = jnp.maximum(jax.lax.dot(m, wt_ref[...],
                                 preferred_element_type=jnp.float32)
                     + b_ref[...], 0.0)
    out_ref[...] = xb + xn


def kernel(node_features, adjacency_matrix, W_in, b_in, W0, b0, W1, b1, W2, b2):
    n = adjacency_matrix.shape[0]
    in_dim = node_features.shape[1]
    d = W_in.shape[0]
    nblk = n // BLK

    prep = pl.pallas_call(
        _prep_kernel,
        grid=(nblk,),
        in_specs=[
            pl.BlockSpec((BLK, n), lambda i: (i, 0)),
            pl.BlockSpec((BLK, in_dim), lambda i: (i, 0)),
            pl.BlockSpec((in_dim, d), lambda i: (0, 0)),
            pl.BlockSpec((1, d), lambda i: (0, 0)),
        ],
        out_specs=(
            pl.BlockSpec((BLK, n), lambda i: (i, 0)),
            pl.BlockSpec((BLK, 1), lambda i: (i, 0)),
            pl.BlockSpec((BLK, d), lambda i: (i, 0)),
        ),
        out_shape=(
            jax.ShapeDtypeStruct((n, n), jnp.bfloat16),
            jax.ShapeDtypeStruct((n, 1), jnp.float32),
            jax.ShapeDtypeStruct((n, d), jnp.float32),
        ),
    )
    abf, dis, x = prep(adjacency_matrix, node_features, W_in.T,
                       b_in.reshape(1, d))

    layer = pl.pallas_call(
        _layer_kernel,
        grid=(nblk,),
        in_specs=[
            pl.BlockSpec((BLK, n), lambda i: (i, 0)),
            pl.BlockSpec((n, d), lambda i: (0, 0)),
            pl.BlockSpec((n, 1), lambda i: (0, 0)),
            pl.BlockSpec((d, d), lambda i: (0, 0)),
            pl.BlockSpec((1, d), lambda i: (0, 0)),
        ],
        out_specs=pl.BlockSpec((BLK, d), lambda i: (i, 0)),
        out_shape=jax.ShapeDtypeStruct((n, d), jnp.float32),
        scratch_shapes=[
            pltpu.VMEM((n, d), jnp.bfloat16),
            pltpu.VMEM((n, d), jnp.bfloat16),
        ],
    )
    for W, b in ((W0, b0), (W1, b1), (W2, b2)):
        x = layer(abf, x, dis, W.T, b.reshape(1, d))
    return x


# speedup vs baseline: 1.5942x; 1.4083x over previous
import jax
import jax.numpy as jnp
from jax.experimental import pallas as pl
from jax.experimental.pallas import tpu as pltpu

BLK = 256


def _mega_kernel(a_ref, nf_ref, wint_ref, bin_ref, ws_ref, bs_ref, out_ref,
                 abf_s, disb_s, x_s, y_s):
    p = pl.program_id(0)
    i = pl.program_id(1)
    r = pl.ds(i * BLK, BLK)

    @pl.when(p == 0)
    def _prep():
        a = a_ref[...]
        abf_s[r, :] = a.astype(jnp.bfloat16)
        deg = jnp.sum(a, axis=1, keepdims=True) + 1.0
        dis = jax.lax.rsqrt(jnp.maximum(deg, 1.0))
        disb_s[r, :] = jnp.broadcast_to(dis, (BLK, disb_s.shape[1]))
        x0 = jnp.maximum(
            jax.lax.dot(nf_ref[...], wint_ref[...],
                        preferred_element_type=jnp.float32) + bin_ref[...],
            0.0)
        x_s[r, :] = x0
        out_ref[...] = x0

    @pl.when(p > 0)
    def _layer():
        @pl.when(i == 0)
        def _scale():
            y_s[...] = (x_s[...] * disb_s[...]).astype(jnp.bfloat16)

        m = jax.lax.dot(abf_s[r, :], y_s[...],
                        preferred_element_type=jnp.float32)
        xb = x_s[r, :]
        db = disb_s[r, :]
        m = db * m + (db * db) * xb
        xn = jnp.maximum(
            jax.lax.dot(m, ws_ref[0], preferred_element_type=jnp.float32)
            + bs_ref[0], 0.0)
        xnew = xb + xn
        x_s[r, :] = xnew
        out_ref[...] = xnew


def kernel(node_features, adjacency_matrix, W_in, b_in, W0, b0, W1, b1, W2, b2):
    n = adjacency_matrix.shape[0]
    in_dim = node_features.shape[1]
    d = W_in.shape[0]
    nblk = n // BLK

    ws = jnp.stack([W0.T, W1.T, W2.T])
    bs = jnp.stack([b0, b1, b2]).reshape(3, 1, d)

    def a_map(p, i):
        return (jnp.where(p == 0, i, nblk - 1), 0)

    def w_map(p, i):
        return (jnp.maximum(p, 1) - 1, 0, 0)

    mega = pl.pallas_call(
        _mega_kernel,
        grid=(4, nblk),
        in_specs=[
            pl.BlockSpec((BLK, n), a_map),
            pl.BlockSpec((BLK, in_dim), a_map),
            pl.BlockSpec((in_dim, d), lambda p, i: (0, 0)),
            pl.BlockSpec((1, d), lambda p, i: (0, 0)),
            pl.BlockSpec((1, d, d), w_map),
            pl.BlockSpec((1, 1, d), w_map),
        ],
        out_specs=pl.BlockSpec((BLK, d), lambda p, i: (i, 0)),
        out_shape=jax.ShapeDtypeStruct((n, d), jnp.float32),
        scratch_shapes=[
            pltpu.VMEM((n, n), jnp.bfloat16),
            pltpu.VMEM((n, d), jnp.float32),
            pltpu.VMEM((n, d), jnp.float32),
            pltpu.VMEM((n, d), jnp.bfloat16),
        ],
        compiler_params=pltpu.CompilerParams(
            vmem_limit_bytes=64 * 1024 * 1024),
    )
    return mega(adjacency_matrix, node_features, W_in.T,
                b_in.reshape(1, d), ws, bs)


# speedup vs baseline: 1.6730x; 1.0494x over previous
import jax
import jax.numpy as jnp
from jax.experimental import pallas as pl
from jax.experimental.pallas import tpu as pltpu

BLK = 256


def _mega_kernel(a_ref, nf_ref, wint_ref, bin_ref, ws_ref, bs_ref, out_ref,
                 abf_s, disb_s, x_s, y_s, acc_s):
    p = pl.program_id(0)
    i = pl.program_id(1)
    nblk = pl.num_programs(1)
    r = pl.ds(i * BLK, BLK)

    @pl.when(p == 0)
    def _prep():
        a = a_ref[...]
        abf_s[r, :] = a.astype(jnp.bfloat16)
        deg = jnp.sum(a, axis=1, keepdims=True) + 1.0
        dis = jax.lax.rsqrt(jnp.maximum(deg, 1.0))
        disb_s[r, :] = jnp.broadcast_to(dis, (BLK, disb_s.shape[1]))
        x0 = jnp.maximum(
            jax.lax.dot(nf_ref[...], wint_ref[...],
                        preferred_element_type=jnp.float32) + bin_ref[...],
            0.0)
        x_s[r, :] = x0

    @pl.when(p > 0)
    def _layer():
        @pl.when(i == 0)
        def _scale():
            y = (x_s[...] * disb_s[...]).astype(jnp.bfloat16)
            y_s[...] = y
            acc_s[...] = y.astype(jnp.float32)

        acc_s[...] += jax.lax.dot(abf_s[:, r], y_s[r, :],
                                  preferred_element_type=jnp.float32)

        @pl.when(i == nblk - 1)
        def _epilogue():
            m = (disb_s[...] * acc_s[...]).astype(jnp.bfloat16)
            xn = jnp.maximum(
                jax.lax.dot(m, ws_ref[0], preferred_element_type=jnp.float32)
                + bs_ref[0], 0.0)
            xnew = x_s[...] + xn
            x_s[...] = xnew

            @pl.when(p == 3)
            def _final():
                out_ref[...] = xnew


def kernel(node_features, adjacency_matrix, W_in, b_in, W0, b0, W1, b1, W2, b2):
    n = adjacency_matrix.shape[0]
    in_dim = node_features.shape[1]
    d = W_in.shape[0]
    nblk = n // BLK

    ws = jnp.stack([W0.T, W1.T, W2.T]).astype(jnp.bfloat16)
    bs = jnp.stack([b0, b1, b2]).reshape(3, 1, d)

    def a_map(p, i):
        return (jnp.where(p == 0, i, nblk - 1), 0)

    def w_map(p, i):
        return (jnp.maximum(p, 1) - 1, 0, 0)

    mega = pl.pallas_call(
        _mega_kernel,
        grid=(4, nblk),
        in_specs=[
            pl.BlockSpec((BLK, n), a_map),
            pl.BlockSpec((BLK, in_dim), a_map),
            pl.BlockSpec((in_dim, d), lambda p, i: (0, 0)),
            pl.BlockSpec((1, d), lambda p, i: (0, 0)),
            pl.BlockSpec((1, d, d), w_map),
            pl.BlockSpec((1, 1, d), w_map),
        ],
        out_specs=pl.BlockSpec((n, d), lambda p, i: (0, 0)),
        out_shape=jax.ShapeDtypeStruct((n, d), jnp.float32),
        scratch_shapes=[
            pltpu.VMEM((n, n), jnp.bfloat16),
            pltpu.VMEM((n, d), jnp.float32),
            pltpu.VMEM((n, d), jnp.float32),
            pltpu.VMEM((n, d), jnp.bfloat16),
            pltpu.VMEM((n, d), jnp.float32),
        ],
        compiler_params=pltpu.CompilerParams(
            vmem_limit_bytes=64 * 1024 * 1024),
    )
    return mega(adjacency_matrix, node_features, W_in.T,
                b_in.reshape(1, d), ws, bs)


# speedup vs baseline: 1.8534x; 1.1078x over previous
import jax
import jax.numpy as jnp
from jax.experimental import pallas as pl
from jax.experimental.pallas import tpu as pltpu

NS = 4
CHUNK = 512
SUB = CHUNK // NS


def _mega_kernel(a0, a1, a2, a3, nf_ref, wint_ref, bin_ref, ws_ref, bs_ref,
                 out_ref, abf_s, disb_s, x_s, y_s, acc_s):
    p = pl.program_id(0)
    i = pl.program_id(1)
    nblk = pl.num_programs(1)
    r = pl.ds(i * CHUNK, CHUNK)
    d = x_s.shape[1]

    @pl.when(p == 0)
    def _prep():
        for k, aref in enumerate((a0, a1, a2, a3)):
            rs = pl.ds(i * CHUNK + k * SUB, SUB)
            a = aref[...]
            abf_s[rs, :] = a.astype(jnp.bfloat16)
            deg = jnp.sum(a, axis=1, keepdims=True) + 1.0
            dis = jax.lax.rsqrt(jnp.maximum(deg, 1.0))
            disb_s[rs, :] = jnp.broadcast_to(dis, (SUB, d))
        x0 = jnp.maximum(
            jax.lax.dot(nf_ref[...], wint_ref[...],
                        preferred_element_type=jnp.float32) + bin_ref[...],
            0.0)
        x_s[r, :] = x0

    @pl.when(p > 0)
    def _layer():
        @pl.when(i == 0)
        def _scale():
            y = (x_s[...] * disb_s[...]).astype(jnp.bfloat16)
            y_s[...] = y
            acc_s[...] = y.astype(jnp.float32)

        acc_s[...] += jax.lax.dot(abf_s[:, r], y_s[r, :],
                                  preferred_element_type=jnp.float32)

        @pl.when(i == nblk - 1)
        def _epilogue():
            m = (disb_s[...] * acc_s[...]).astype(jnp.bfloat16)
            xn = jnp.maximum(
                jax.lax.dot(m, ws_ref[0], preferred_element_type=jnp.float32)
                + bs_ref[0], 0.0)
            xnew = x_s[...] + xn
            x_s[...] = xnew

            @pl.when(p == 3)
            def _final():
                out_ref[...] = xnew


def kernel(node_features, adjacency_matrix, W_in, b_in, W0, b0, W1, b1, W2, b2):
    n = adjacency_matrix.shape[0]
    in_dim = node_features.shape[1]
    d = W_in.shape[0]
    nblk = n // CHUNK

    ws = jnp.stack([W0.T, W1.T, W2.T]).astype(jnp.bfloat16)
    bs = jnp.stack([b0, b1, b2]).reshape(3, 1, d)

    def w_map(p, i):
        return (jnp.maximum(p, 1) - 1, 0, 0)

    mega = pl.pallas_call(
        _mega_kernel,
        grid=(4, nblk),
        in_specs=[
            pl.BlockSpec(
                (SUB, n),
                lambda p, i, k=k: (jnp.where(p == 0, NS * i + k,
                                             NS * (nblk - 1) + k), 0))
            for k in range(NS)
        ] + [
            pl.BlockSpec((CHUNK, in_dim),
                         lambda p, i: (jnp.where(p == 0, i, nblk - 1), 0)),
            pl.BlockSpec((in_dim, d), lambda p, i: (0, 0)),
            pl.BlockSpec((1, d), lambda p, i: (0, 0)),
            pl.BlockSpec((1, d, d), w_map),
            pl.BlockSpec((1, 1, d), w_map),
        ],
        out_specs=pl.BlockSpec((n, d), lambda p, i: (0, 0)),
        out_shape=jax.ShapeDtypeStruct((n, d), jnp.float32),
        scratch_shapes=[
            pltpu.VMEM((n, n), jnp.bfloat16),
            pltpu.VMEM((n, d), jnp.float32),
            pltpu.VMEM((n, d), jnp.float32),
            pltpu.VMEM((n, d), jnp.bfloat16),
            pltpu.VMEM((n, d), jnp.float32),
        ],
        compiler_params=pltpu.CompilerParams(
            vmem_limit_bytes=64 * 1024 * 1024),
    )
    return mega(adjacency_matrix, adjacency_matrix, adjacency_matrix,
                adjacency_matrix, node_features, W_in.T,
                b_in.reshape(1, d), ws, bs)
